# Initial kernel scaffold; baseline (speedup 1.0000x reference)
import jax
import jax.numpy as jnp
from jax.experimental import pallas as pl


def kernel(x):
    raise NotImplementedError("write your pallas kernel here")


# speedup vs baseline: 1.9369x; 1.9369x over previous
import jax
import jax.numpy as jnp
from jax import lax
from jax.experimental import pallas as pl

_MERGED = 100


def _body(x_ref, out_ref, tok_ref):
    x = x_ref[0]
    C, HW = x.shape
    FW = 32
    G = HW // 4
    L = HW - 3 * _MERGED
    LP = 768

    f32 = jnp.float32
    t_row = lax.broadcasted_iota(jnp.int32, (1, HW), 1)
    col = t_row % FW
    row = t_row // FW
    col_even = (col & 1) == 0
    row_even = (row & 1) == 0

    x1 = jnp.roll(x, 1, axis=1)
    a = jnp.where(col_even, x, x1)
    refv = jnp.where(row_even, a, jnp.roll(a, FW, axis=1))
    d = jnp.abs(x - refv)
    tsum = jnp.sum(d, axis=0, keepdims=True)

    tc = lax.broadcasted_iota(jnp.int32, (HW, G), 0)
    g_of_t = ((tc // FW) >> 1) * (FW // 2) + ((tc % FW) >> 1)
    M = (g_of_t == lax.broadcasted_iota(jnp.int32, (HW, G), 1)).astype(f32)
    s = lax.dot_general(tsum, M, (((1,), (0,)), ((), ())),
                        preferred_element_type=f32,
                        precision=lax.Precision.HIGHEST)

    S_g = jnp.broadcast_to(s, (G, G))
    S_j = jnp.transpose(S_g)
    j_i = lax.broadcasted_iota(jnp.int32, (G, G), 0)
    g_i = lax.broadcasted_iota(jnp.int32, (G, G), 1)
    cmp = (S_j < S_g) | ((S_j == S_g) & (j_i < g_i))
    rank = jnp.sum(cmp.astype(jnp.int32), axis=0, keepdims=True)
    keep_group = (rank >= _MERGED).astype(f32)

    kgt = lax.dot_general(keep_group, M, (((1,), (1,)), ((), ())),
                          preferred_element_type=f32)
    keep = (row_even & col_even) | (kgt > 0.5)
    keep_f = keep.astype(f32)

    lt = (lax.broadcasted_iota(jnp.int32, (HW, HW), 0)
          < lax.broadcasted_iota(jnp.int32, (HW, HW), 1)).astype(f32)
    pos = lax.dot_general(keep_f, lt, (((1,), (0,)), ((), ())),
                          preferred_element_type=f32)

    l_i = lax.broadcasted_iota(jnp.int32, (LP, HW), 0).astype(f32)
    P = ((jnp.broadcast_to(pos, (LP, HW)) == l_i)
         & jnp.broadcast_to(keep, (LP, HW))).astype(f32)

    bf16 = jnp.bfloat16
    x_hi = x.astype(bf16)
    r1 = x - x_hi.astype(f32)
    x_mid = r1.astype(bf16)
    x_lo = (r1 - x_mid.astype(f32)).astype(bf16)
    Pb = P.astype(bf16)
    dims = (((1,), (1,)), ((), ()))
    out = (lax.dot_general(Pb, x_hi, dims, preferred_element_type=f32)
           + lax.dot_general(Pb, x_mid, dims, preferred_element_type=f32)
           + lax.dot_general(Pb, x_lo, dims, preferred_element_type=f32))
    out_ref[0] = out[:L, :]

    tokrow = lax.dot_general(t_row.astype(f32), P, (((1,), (1,)), ((), ())),
                             preferred_element_type=f32,
                             precision=lax.Precision.HIGHEST)
    tok_ref[0] = tokrow[:, :L].astype(jnp.int32)


def kernel(x):
    B, C, FH, FW = x.shape
    HW = FH * FW
    L = HW - 3 * _MERGED
    xf = x.reshape(B, C, HW)
    out, tok = pl.pallas_call(
        _body,
        grid=(B,),
        in_specs=[pl.BlockSpec((1, C, HW), lambda b: (b, 0, 0))],
        out_specs=[
            pl.BlockSpec((1, L, C), lambda b: (b, 0, 0)),
            pl.BlockSpec((1, 1, L), lambda b: (b, 0, 0)),
        ],
        out_shape=[
            jax.ShapeDtypeStruct((B, L, C), jnp.float32),
            jax.ShapeDtypeStruct((B, 1, L), jnp.int32),
        ],
    )(xf)
    return (out, tok.reshape(B, L))


# speedup vs baseline: 2.2429x; 1.1580x over previous
import jax
import jax.numpy as jnp
from jax import lax
from jax.experimental import pallas as pl

_MERGED = 100
_TL = 128
_W = 512


def _body(m_ref, lt_ref, x_ref, out_ref, tok_ref):
    x = x_ref[0]
    C, HW = x.shape
    FW = 32
    G = HW // 4
    L = HW - 3 * _MERGED
    LP = 768

    f32 = jnp.float32
    bf16 = jnp.bfloat16
    t_row = lax.broadcasted_iota(jnp.int32, (1, HW), 1)
    col = t_row % FW
    row = t_row // FW
    col_even = (col & 1) == 0
    row_even = (row & 1) == 0

    x1 = jnp.roll(x, 1, axis=1)
    a = jnp.where(col_even, x, x1)
    refv = jnp.where(row_even, a, jnp.roll(a, FW, axis=1))
    d = jnp.abs(x - refv)
    tsum = jnp.sum(d, axis=0, keepdims=True)

    M = m_ref[...]
    s = lax.dot_general(tsum, M, (((1,), (0,)), ((), ())),
                        preferred_element_type=f32,
                        precision=lax.Precision.HIGHEST)

    S_g = jnp.broadcast_to(s, (G, G))
    S_j = jnp.transpose(S_g)
    j_i = lax.broadcasted_iota(jnp.int32, (G, G), 0)
    g_i = lax.broadcasted_iota(jnp.int32, (G, G), 1)
    cmp = (S_j < S_g) | ((S_j == S_g) & (j_i < g_i))
    rank = jnp.sum(cmp.astype(jnp.int32), axis=0, keepdims=True)
    keep_group = (rank >= _MERGED).astype(f32)

    kgt = lax.dot_general(keep_group, M, (((1,), (1,)), ((), ())),
                          preferred_element_type=f32)
    keep = (row_even & col_even) | (kgt > 0.5)
    keep_f = keep.astype(f32)

    pos = lax.dot_general(keep_f, lt_ref[...], (((1,), (0,)), ((), ())),
                          preferred_element_type=f32)

    x_hi = x.astype(bf16)
    r1 = x - x_hi.astype(f32)
    x_mid = r1.astype(bf16)
    x_lo = (r1 - x_mid.astype(f32)).astype(bf16)
    ta = (t_row // 256).astype(bf16)
    tb = (t_row % 256).astype(bf16)

    dims = (((1,), (1,)), ((), ()))
    for k in range(LP // _TL):
        l0 = k * _TL
        t0 = min(l0, HW - _W)
        sl = slice(t0, t0 + _W)
        pos_w = pos[:, sl]
        keep_w = keep[:, sl]
        l_i = (l0 + lax.broadcasted_iota(jnp.int32, (_TL, _W), 0)).astype(f32)
        Pb = ((jnp.broadcast_to(pos_w, (_TL, _W)) == l_i)
              & jnp.broadcast_to(keep_w, (_TL, _W))).astype(bf16)
        out_k = (lax.dot_general(Pb, x_hi[:, sl], dims, preferred_element_type=f32)
                 + lax.dot_general(Pb, x_mid[:, sl], dims, preferred_element_type=f32)
                 + lax.dot_general(Pb, x_lo[:, sl], dims, preferred_element_type=f32))
        tok_k = (256.0 * lax.dot_general(ta[:, sl], Pb, dims, preferred_element_type=f32)
                 + lax.dot_general(tb[:, sl], Pb, dims, preferred_element_type=f32))
        n = min(_TL, L - l0)
        out_ref[0, l0:l0 + n, :] = out_k[:n, :]
        tok_ref[0, :, l0:l0 + n] = tok_k[:, :n].astype(jnp.int32)


def kernel(x):
    B, C, FH, FW = x.shape
    HW = FH * FW
    G = HW // 4
    L = HW - 3 * _MERGED
    xf = x.reshape(B, C, HW)

    t = jnp.arange(HW, dtype=jnp.int32)
    g_of_t = (t // FW // 2) * (FW // 2) + (t % FW) // 2
    m_const = (g_of_t[:, None] == jnp.arange(G, dtype=jnp.int32)[None, :]
               ).astype(jnp.float32)
    lt_const = (t[:, None] < t[None, :]).astype(jnp.float32)

    out, tok = pl.pallas_call(
        _body,
        grid=(B,),
        in_specs=[
            pl.BlockSpec((HW, G), lambda b: (0, 0)),
            pl.BlockSpec((HW, HW), lambda b: (0, 0)),
            pl.BlockSpec((1, C, HW), lambda b: (b, 0, 0)),
        ],
        out_specs=[
            pl.BlockSpec((1, L, C), lambda b: (b, 0, 0)),
            pl.BlockSpec((1, 1, L), lambda b: (b, 0, 0)),
        ],
        out_shape=[
            jax.ShapeDtypeStruct((B, L, C), jnp.float32),
            jax.ShapeDtypeStruct((B, 1, L), jnp.int32),
        ],
    )(m_const, lt_const, xf)
    return (out, tok.reshape(B, L))
